# baseline (speedup 1.0000x reference)
import functools

import jax
import jax.numpy as jnp
from jax import lax
from jax.experimental import pallas as pl
from jax.experimental.pallas import tpu as pltpu
from jax.experimental.pallas import tpu_sc as plsc

_B = 16384
_D = 64
_NC = 2
_NS = 16
_NW = _NC * _NS
_BPW = _B // _NW
_CH = 128
_NCHUNK = _BPW // _CH
_IDX_ROWS = _B // _CH


def _shift_idx(raw, div):
    for k in range(_NCHUNK):
        for s in range(_CH // 16):
            v = raw[k, pl.ds(s * 16, 16)]
            div[k, pl.ds(s * 16, 16)] = lax.shift_right_logical(v, 1)


def _body(ph, pr, pt, nh, nr, nt, ent2, rel2, pos_out, neg_out,
          rh, rr, rt, dh, dr, dt, bh, br, bt, out_v, sem):
    wid = lax.axis_index("s") * _NC + lax.axis_index("c")

    def do_term(hi, ri, ti, out_hbm):
        pltpu.sync_copy(hi.at[pl.ds(wid * _NCHUNK, _NCHUNK)], rh)
        pltpu.sync_copy(ri.at[pl.ds(wid * _NCHUNK, _NCHUNK)], rr)
        pltpu.sync_copy(ti.at[pl.ds(wid * _NCHUNK, _NCHUNK)], rt)
        _shift_idx(rh, dh)
        _shift_idx(rr, dr)
        _shift_idx(rt, dt)
        for c in range(_NCHUNK):
            cp1 = pltpu.async_copy(ent2.at[dh.at[c]], bh, sem)
            cp2 = pltpu.async_copy(rel2.at[dr.at[c]], br, sem)
            cp3 = pltpu.async_copy(ent2.at[dt.at[c]], bt, sem)
            cp1.wait()
            cp2.wait()
            cp3.wait()
            for g in range(_CH // 16):
                rows = lax.iota(jnp.int32, 16) + (g * 16)
                base_h = (rh[c, pl.ds(g * 16, 16)] & 1) * _D
                base_r = (rr[c, pl.ds(g * 16, 16)] & 1) * _D
                base_t = (rt[c, pl.ds(g * 16, 16)] & 1) * _D

                def jbody(j, acc):
                    h = plsc.load_gather(bh, [rows, base_h + j])
                    r = plsc.load_gather(br, [rows, base_r + j])
                    t = plsc.load_gather(bt, [rows, base_t + j])
                    d = h + r - t
                    return acc + d * d

                acc = lax.fori_loop(0, _D, jbody,
                                    jnp.zeros((16,), jnp.float32))
                out_v[pl.ds(c * _CH + g * 16, 16)] = acc
        pltpu.sync_copy(out_v, out_hbm.at[pl.ds(wid * _BPW, _BPW)])

    do_term(ph, pr, pt, pos_out)
    do_term(nh, nr, nt, neg_out)


@functools.partial(jax.jit)
def kernel(ph, pr, pt, nh, nr, nt, ent_embed, rel_embed):
    idxs = [x.astype(jnp.int32).reshape(_IDX_ROWS, _CH)
            for x in (ph, pr, pt, nh, nr, nt)]
    ent2 = ent_embed.reshape(ent_embed.shape[0] // 2, 2 * _D)
    rel2 = rel_embed.reshape(rel_embed.shape[0] // 2, 2 * _D)
    mesh = plsc.VectorSubcoreMesh(core_axis_name="c", subcore_axis_name="s",
                                  num_cores=_NC, num_subcores=_NS)
    f = pl.kernel(
        _body,
        out_type=(jax.ShapeDtypeStruct((_B,), jnp.float32),
                  jax.ShapeDtypeStruct((_B,), jnp.float32)),
        mesh=mesh,
        scratch_types=[
            pltpu.VMEM((_NCHUNK, _CH), jnp.int32),
            pltpu.VMEM((_NCHUNK, _CH), jnp.int32),
            pltpu.VMEM((_NCHUNK, _CH), jnp.int32),
            pltpu.VMEM((_NCHUNK, _CH), jnp.int32),
            pltpu.VMEM((_NCHUNK, _CH), jnp.int32),
            pltpu.VMEM((_NCHUNK, _CH), jnp.int32),
            pltpu.VMEM((_CH, 2 * _D), jnp.float32),
            pltpu.VMEM((_CH, 2 * _D), jnp.float32),
            pltpu.VMEM((_CH, 2 * _D), jnp.float32),
            pltpu.VMEM((_BPW,), jnp.float32),
            pltpu.SemaphoreType.DMA,
        ],
        compiler_params=pltpu.CompilerParams(needs_layout_passes=False,
                                             use_tc_tiling_on_sc=True),
    )
    return f(*idxs, ent2, rel2)


# speedup vs baseline: 1.2429x; 1.2429x over previous
import functools

import jax
import jax.numpy as jnp
from jax import lax
from jax.experimental import pallas as pl
from jax.experimental.pallas import tpu as pltpu
from jax.experimental.pallas import tpu_sc as plsc

_B = 16384
_D = 64
_NC = 2
_NS = 16
_NW = _NC * _NS
_BPW = _B // _NW
_IC = 128
_NIR = _BPW // _IC
_IDX_ROWS = _B // _IC
_G = 16


def _body(ph, pr, pt, nh, nr, nt, ent, rel2, pos_out, neg_out,
          rh, rr, rt, dr, bh, bt, brl, out_v, sem, sem_r):
    wid = lax.axis_index("s") * _NC + lax.axis_index("c")

    def do_term(hi, ri, ti, out_hbm):
        pltpu.sync_copy(hi.at[pl.ds(wid * _NIR, _NIR)], rh)
        pltpu.sync_copy(ri.at[pl.ds(wid * _NIR, _NIR)], rr)
        pltpu.sync_copy(ti.at[pl.ds(wid * _NIR, _NIR)], rt)
        for k in range(_NIR):
            for s in range(_IC // 16):
                v = rr[k, pl.ds(s * 16, 16)]
                dr[k, pl.ds(s * 16, 16)] = lax.shift_right_logical(v, 1)
        for c in range(_NIR):
            cpr = pltpu.async_copy(rel2.at[dr.at[c]], brl, sem_r)
            cpr.wait()

            def gbody(g, _):
                v_h = rh[c, pl.ds(g * _G, _G)]
                v_t = rt[c, pl.ds(g * _G, _G)]
                cps = []
                for i in range(_G):
                    sh = pl.multiple_of(
                        (lax.shift_right_logical(v_h[i], 3) * 8).astype(
                            jnp.int32), 8)
                    st = pl.multiple_of(
                        (lax.shift_right_logical(v_t[i], 3) * 8).astype(
                            jnp.int32), 8)
                    cps.append(pltpu.async_copy(
                        ent.at[pl.ds(sh, 8)], bh.at[i], sem))
                    cps.append(pltpu.async_copy(
                        ent.at[pl.ds(st, 8)], bt.at[i], sem))
                for cp in cps:
                    cp.wait()
                rows = lax.iota(jnp.int32, 16)
                crows = rows + (g * _G)
                k_h = v_h & 7
                k_t = v_t & 7
                base_r = (rr[c, pl.ds(g * _G, _G)] & 1) * _D

                def jbody(j, acc):
                    jv = lax.broadcast(j, (16,))
                    h = plsc.load_gather(bh, [rows, k_h, jv])
                    t = plsc.load_gather(bt, [rows, k_t, jv])
                    r = plsc.load_gather(brl, [crows, base_r + j])
                    d = h + r - t
                    return acc + d * d

                acc = lax.fori_loop(0, _D, jbody,
                                    jnp.zeros((16,), jnp.float32))
                out_v[pl.ds(c * _IC + g * _G, 16)] = acc
                return 0

            lax.fori_loop(0, _IC // _G, gbody, 0)
        pltpu.sync_copy(out_v, out_hbm.at[pl.ds(wid * _BPW, _BPW)])

    do_term(ph, pr, pt, pos_out)
    do_term(nh, nr, nt, neg_out)


@functools.partial(jax.jit)
def kernel(ph, pr, pt, nh, nr, nt, ent_embed, rel_embed):
    idxs = [x.astype(jnp.int32).reshape(_IDX_ROWS, _IC)
            for x in (ph, pr, pt, nh, nr, nt)]
    rel2 = rel_embed.reshape(rel_embed.shape[0] // 2, 2 * _D)
    mesh = plsc.VectorSubcoreMesh(core_axis_name="c", subcore_axis_name="s",
                                  num_cores=_NC, num_subcores=_NS)
    f = pl.kernel(
        _body,
        out_type=(jax.ShapeDtypeStruct((_B,), jnp.float32),
                  jax.ShapeDtypeStruct((_B,), jnp.float32)),
        mesh=mesh,
        scratch_types=[
            pltpu.VMEM((_NIR, _IC), jnp.int32),
            pltpu.VMEM((_NIR, _IC), jnp.int32),
            pltpu.VMEM((_NIR, _IC), jnp.int32),
            pltpu.VMEM((_NIR, _IC), jnp.int32),
            pltpu.VMEM((_G, 8, _D), jnp.float32),
            pltpu.VMEM((_G, 8, _D), jnp.float32),
            pltpu.VMEM((_IC, 2 * _D), jnp.float32),
            pltpu.VMEM((_BPW,), jnp.float32),
            pltpu.SemaphoreType.DMA,
            pltpu.SemaphoreType.DMA,
        ],
        compiler_params=pltpu.CompilerParams(needs_layout_passes=False,
                                             use_tc_tiling_on_sc=True),
    )
    return f(*idxs, ent_embed, rel2)


# speedup vs baseline: 1.8791x; 1.5119x over previous
import functools

import jax
import jax.numpy as jnp
from jax import lax
from jax.experimental import pallas as pl
from jax.experimental.pallas import tpu as pltpu
from jax.experimental.pallas import tpu_sc as plsc

_B = 16384
_D = 64
_NC = 2
_NS = 16
_NW = _NC * _NS
_BPW = _B // _NW
_IC = 128
_NIR = _BPW // _IC
_IDX_ROWS = _B // _IC
_G = 16
_NG = _IC // _G


def _body(ph, pr, pt, nh, nr, nt, ent3, rel2, pos_out, neg_out,
          rh, rr, rt, dr, bh0, bh1, bt0, bt1, brl, out_v,
          sem0, sem1, sem_r):
    wid = lax.axis_index("s") * _NC + lax.axis_index("c")
    sets = ((bh0, bt0, sem0), (bh1, bt1, sem1))

    def fire(c, g, b):
        bh, bt, sem = sets[b]
        v_h = rh[c, pl.ds(g * _G, _G)]
        v_t = rt[c, pl.ds(g * _G, _G)]
        for i in range(_G):
            sh = lax.shift_right_logical(v_h[i], 3)
            st = lax.shift_right_logical(v_t[i], 3)
            pltpu.async_copy(ent3.at[sh], bh.at[i], sem)
            pltpu.async_copy(ent3.at[st], bt.at[i], sem)

    def drain(b):
        bh, bt, sem = sets[b]
        for i in range(_G):
            pltpu.make_async_copy(ent3.at[0], bh.at[i], sem).wait()
            pltpu.make_async_copy(ent3.at[0], bt.at[i], sem).wait()

    def compute(c, g, b):
        bh, bt, _ = sets[b]
        v_h = rh[c, pl.ds(g * _G, _G)]
        v_t = rt[c, pl.ds(g * _G, _G)]
        rows = lax.iota(jnp.int32, 16)
        crows = rows + (g * _G)
        k_h = v_h & 7
        k_t = v_t & 7
        base_r = (rr[c, pl.ds(g * _G, _G)] & 1) * _D

        def jbody(j, acc):
            jv = lax.broadcast(j, (16,))
            h = plsc.load_gather(bh, [rows, k_h, jv])
            t = plsc.load_gather(bt, [rows, k_t, jv])
            r = plsc.load_gather(brl, [crows, base_r + j])
            d = h + r - t
            return acc + d * d

        acc = lax.fori_loop(0, _D, jbody, jnp.zeros((16,), jnp.float32))
        out_v[pl.ds(c * _IC + g * _G, 16)] = acc

    def do_term(hi, ri, ti, out_hbm):
        pltpu.sync_copy(hi.at[pl.ds(wid * _NIR, _NIR)], rh)
        pltpu.sync_copy(ri.at[pl.ds(wid * _NIR, _NIR)], rr)
        pltpu.sync_copy(ti.at[pl.ds(wid * _NIR, _NIR)], rt)
        for k in range(_NIR):
            for s in range(_IC // 16):
                v = rr[k, pl.ds(s * 16, 16)]
                dr[k, pl.ds(s * 16, 16)] = lax.shift_right_logical(v, 1)
        for c in range(_NIR):
            cpr = pltpu.async_copy(rel2.at[dr.at[c]], brl, sem_r)
            fire(c, 0, 0)
            fire(c, 1, 1)
            cpr.wait()

            def qbody(q, _):
                for b in range(2):
                    g = q * 2 + b
                    drain(b)
                    compute(c, g, b)

                    @pl.when(g + 2 < _NG)
                    def _():
                        fire(c, g + 2, b)
                return 0

            lax.fori_loop(0, _NG // 2, qbody, 0)
        pltpu.sync_copy(out_v, out_hbm.at[pl.ds(wid * _BPW, _BPW)])

    do_term(ph, pr, pt, pos_out)
    do_term(nh, nr, nt, neg_out)


@functools.partial(jax.jit)
def kernel(ph, pr, pt, nh, nr, nt, ent_embed, rel_embed):
    idxs = [x.astype(jnp.int32).reshape(_IDX_ROWS, _IC)
            for x in (ph, pr, pt, nh, nr, nt)]
    ent3 = ent_embed.reshape(ent_embed.shape[0] // 8, 8, _D)
    rel2 = rel_embed.reshape(rel_embed.shape[0] // 2, 2 * _D)
    mesh = plsc.VectorSubcoreMesh(core_axis_name="c", subcore_axis_name="s",
                                  num_cores=_NC, num_subcores=_NS)
    f = pl.kernel(
        _body,
        out_type=(jax.ShapeDtypeStruct((_B,), jnp.float32),
                  jax.ShapeDtypeStruct((_B,), jnp.float32)),
        mesh=mesh,
        scratch_types=[
            pltpu.VMEM((_NIR, _IC), jnp.int32),
            pltpu.VMEM((_NIR, _IC), jnp.int32),
            pltpu.VMEM((_NIR, _IC), jnp.int32),
            pltpu.VMEM((_NIR, _IC), jnp.int32),
            pltpu.VMEM((_G, 8, _D), jnp.float32),
            pltpu.VMEM((_G, 8, _D), jnp.float32),
            pltpu.VMEM((_G, 8, _D), jnp.float32),
            pltpu.VMEM((_G, 8, _D), jnp.float32),
            pltpu.VMEM((_IC, 2 * _D), jnp.float32),
            pltpu.VMEM((_BPW,), jnp.float32),
            pltpu.SemaphoreType.DMA,
            pltpu.SemaphoreType.DMA,
            pltpu.SemaphoreType.DMA,
        ],
        compiler_params=pltpu.CompilerParams(needs_layout_passes=False,
                                             use_tc_tiling_on_sc=True),
    )
    return f(*idxs, ent3, rel2)


# speedup vs baseline: 1.9260x; 1.0250x over previous
import functools

import jax
import jax.numpy as jnp
from jax import lax
from jax.experimental import pallas as pl
from jax.experimental.pallas import tpu as pltpu
from jax.experimental.pallas import tpu_sc as plsc

_B = 16384
_D = 64
_NC = 2
_NS = 16
_NW = _NC * _NS
_BPW = _B // _NW
_IC = 128
_NIR = _BPW // _IC
_IDX_ROWS = _B // _IC
_G = 16
_NG = _IC // _G


def _body(ph, pr, pt, nh, nr, nt, ent3, rel2, pos_out, neg_out,
          rh, rr, rt, dr, bh0, bh1, bt0, bt1, brl, out_v,
          sem0, sem1, sem_r):
    wid = lax.axis_index("s") * _NC + lax.axis_index("c")
    sets = ((bh0, bt0, sem0), (bh1, bt1, sem1))

    def fire(c, g, b):
        bh, bt, sem = sets[b]
        v_h = rh[c, pl.ds(g * _G, _G)]
        v_t = rt[c, pl.ds(g * _G, _G)]
        for i in range(_G):
            sh = lax.shift_right_logical(v_h[i], 3)
            st = lax.shift_right_logical(v_t[i], 3)
            pltpu.async_copy(ent3.at[sh], bh.at[i], sem)
            pltpu.async_copy(ent3.at[st], bt.at[i], sem)

    def drain(b):
        bh, bt, sem = sets[b]
        pltpu.make_async_copy(ent3.at[pl.ds(0, _G)], bh, sem).wait()
        pltpu.make_async_copy(ent3.at[pl.ds(0, _G)], bt, sem).wait()

    def compute(c, g, b):
        bh, bt, _ = sets[b]
        v_h = rh[c, pl.ds(g * _G, _G)]
        v_t = rt[c, pl.ds(g * _G, _G)]
        rows = lax.iota(jnp.int32, 16)
        crows = rows + (g * _G)
        k_h = v_h & 7
        k_t = v_t & 7
        base_r = (rr[c, pl.ds(g * _G, _G)] & 1) * _D

        def jbody(j4, acc):
            for u in range(4):
                j = j4 * 4 + u
                jv = lax.broadcast(j, (16,))
                h = plsc.load_gather(bh, [rows, k_h, jv])
                t = plsc.load_gather(bt, [rows, k_t, jv])
                r = plsc.load_gather(brl, [crows, base_r + j])
                d = h + r - t
                acc = acc + d * d
            return acc

        acc = lax.fori_loop(0, _D // 4, jbody, jnp.zeros((16,), jnp.float32))
        out_v[pl.ds(c * _IC + g * _G, 16)] = acc

    def do_term(hi, ri, ti, out_hbm):
        pltpu.sync_copy(hi.at[pl.ds(wid * _NIR, _NIR)], rh)
        pltpu.sync_copy(ri.at[pl.ds(wid * _NIR, _NIR)], rr)
        pltpu.sync_copy(ti.at[pl.ds(wid * _NIR, _NIR)], rt)
        for k in range(_NIR):
            for s in range(_IC // 16):
                v = rr[k, pl.ds(s * 16, 16)]
                dr[k, pl.ds(s * 16, 16)] = lax.shift_right_logical(v, 1)
        for c in range(_NIR):
            cpr = pltpu.async_copy(rel2.at[dr.at[c]], brl, sem_r)
            fire(c, 0, 0)
            fire(c, 1, 1)
            cpr.wait()

            def qbody(q, _):
                for b in range(2):
                    g = q * 2 + b
                    drain(b)
                    compute(c, g, b)

                    @pl.when(g + 2 < _NG)
                    def _():
                        fire(c, g + 2, b)
                return 0

            lax.fori_loop(0, _NG // 2, qbody, 0)
        pltpu.sync_copy(out_v, out_hbm.at[pl.ds(wid * _BPW, _BPW)])

    do_term(ph, pr, pt, pos_out)
    do_term(nh, nr, nt, neg_out)


@functools.partial(jax.jit)
def kernel(ph, pr, pt, nh, nr, nt, ent_embed, rel_embed):
    idxs = [x.astype(jnp.int32).reshape(_IDX_ROWS, _IC)
            for x in (ph, pr, pt, nh, nr, nt)]
    ent3 = ent_embed.reshape(ent_embed.shape[0] // 8, 8, _D)
    rel2 = rel_embed.reshape(rel_embed.shape[0] // 2, 2 * _D)
    mesh = plsc.VectorSubcoreMesh(core_axis_name="c", subcore_axis_name="s",
                                  num_cores=_NC, num_subcores=_NS)
    f = pl.kernel(
        _body,
        out_type=(jax.ShapeDtypeStruct((_B,), jnp.float32),
                  jax.ShapeDtypeStruct((_B,), jnp.float32)),
        mesh=mesh,
        scratch_types=[
            pltpu.VMEM((_NIR, _IC), jnp.int32),
            pltpu.VMEM((_NIR, _IC), jnp.int32),
            pltpu.VMEM((_NIR, _IC), jnp.int32),
            pltpu.VMEM((_NIR, _IC), jnp.int32),
            pltpu.VMEM((_G, 8, _D), jnp.float32),
            pltpu.VMEM((_G, 8, _D), jnp.float32),
            pltpu.VMEM((_G, 8, _D), jnp.float32),
            pltpu.VMEM((_G, 8, _D), jnp.float32),
            pltpu.VMEM((_IC, 2 * _D), jnp.float32),
            pltpu.VMEM((_BPW,), jnp.float32),
            pltpu.SemaphoreType.DMA,
            pltpu.SemaphoreType.DMA,
            pltpu.SemaphoreType.DMA,
        ],
        compiler_params=pltpu.CompilerParams(needs_layout_passes=False,
                                             use_tc_tiling_on_sc=True),
    )
    return f(*idxs, ent3, rel2)


# speedup vs baseline: 2.1631x; 1.1231x over previous
import functools

import jax
import jax.numpy as jnp
from jax import lax
from jax.experimental import pallas as pl
from jax.experimental.pallas import tpu as pltpu
from jax.experimental.pallas import tpu_sc as plsc

_B = 16384
_D = 64
_NC = 2
_NS = 16
_NW = _NC * _NS
_BPW = _B // _NW
_G = 16
_NG = _BPW // _G
_NB = 3
_NQ = 10


def _body(ph, pr, pt, nh, nr, nt, ent3, rel2, pos_out, neg_out,
          rh, rr, rt, dr, out_v, *setargs):
    wid = lax.axis_index("s") * _NC + lax.axis_index("c")
    sets = [(setargs[4 * b], setargs[4 * b + 1], setargs[4 * b + 2],
             setargs[4 * b + 3]) for b in range(_NB)]

    def fire(g, b):
        bh, bt, brl, sem = sets[b]
        v_h = rh[pl.ds(g * _G, _G)]
        v_t = rt[pl.ds(g * _G, _G)]
        for i in range(_G):
            pltpu.async_copy(ent3.at[lax.shift_right_logical(v_h[i], 3)],
                             bh.at[i], sem)
            pltpu.async_copy(ent3.at[lax.shift_right_logical(v_t[i], 3)],
                             bt.at[i], sem)
        pltpu.async_copy(rel2.at[dr.at[pl.ds(g * _G, _G)]], brl, sem)

    def drain(b):
        bh, bt, brl, sem = sets[b]
        pltpu.make_async_copy(ent3.at[pl.ds(0, _G)], bh, sem).wait()
        pltpu.make_async_copy(ent3.at[pl.ds(0, _G)], bt, sem).wait()
        pltpu.make_async_copy(rel2.at[pl.ds(0, _G)], brl, sem).wait()

    def compute(g, b):
        bh, bt, brl, _ = sets[b]
        v_h = rh[pl.ds(g * _G, _G)]
        v_t = rt[pl.ds(g * _G, _G)]
        rows = lax.iota(jnp.int32, 16)
        k_h = v_h & 7
        k_t = v_t & 7
        base_r = (rr[pl.ds(g * _G, _G)] & 1) * _D

        def jbody(j4, acc):
            for u in range(4):
                j = j4 * 4 + u
                jv = lax.broadcast(j, (16,))
                h = plsc.load_gather(bh, [rows, k_h, jv])
                t = plsc.load_gather(bt, [rows, k_t, jv])
                r = plsc.load_gather(brl, [rows, base_r + j])
                d = h + r - t
                acc = acc + d * d
            return acc

        acc = lax.fori_loop(0, _D // 4, jbody, jnp.zeros((16,), jnp.float32))
        out_v[pl.ds(g * _G, 16)] = acc

    def do_term(hi, ri, ti, out_hbm):
        pltpu.sync_copy(hi.at[wid], rh)
        pltpu.sync_copy(ri.at[wid], rr)
        pltpu.sync_copy(ti.at[wid], rt)
        for s in range(_BPW // 16):
            v = rr[pl.ds(s * 16, 16)]
            dr[pl.ds(s * 16, 16)] = lax.shift_right_logical(v, 1)
        for b in range(_NB):
            fire(b, b)

        def qbody(q, _):
            for b in range(_NB):
                g = q * _NB + b
                drain(b)
                compute(g, b)

                @pl.when(g + _NB < _NG)
                def _():
                    fire(g + _NB, b)
            return 0

        lax.fori_loop(0, _NQ, qbody, 0)
        for g in range(_NQ * _NB, _NG):
            b = g % _NB
            drain(b)
            compute(g, b)
        pltpu.sync_copy(out_v, out_hbm.at[pl.ds(wid * _BPW, _BPW)])

    do_term(ph, pr, pt, pos_out)
    do_term(nh, nr, nt, neg_out)


@functools.partial(jax.jit)
def kernel(ph, pr, pt, nh, nr, nt, ent_embed, rel_embed):
    idxs = [x.astype(jnp.int32).reshape(_NW, _BPW)
            for x in (ph, pr, pt, nh, nr, nt)]
    ent3 = ent_embed.reshape(ent_embed.shape[0] // 8, 8, _D)
    rel2 = rel_embed.reshape(rel_embed.shape[0] // 2, 2 * _D)
    mesh = plsc.VectorSubcoreMesh(core_axis_name="c", subcore_axis_name="s",
                                  num_cores=_NC, num_subcores=_NS)
    set_scratch = []
    for _ in range(_NB):
        set_scratch += [
            pltpu.VMEM((_G, 8, _D), jnp.float32),
            pltpu.VMEM((_G, 8, _D), jnp.float32),
            pltpu.VMEM((_G, 2 * _D), jnp.float32),
            pltpu.SemaphoreType.DMA,
        ]
    f = pl.kernel(
        _body,
        out_type=(jax.ShapeDtypeStruct((_B,), jnp.float32),
                  jax.ShapeDtypeStruct((_B,), jnp.float32)),
        mesh=mesh,
        scratch_types=[
            pltpu.VMEM((_BPW,), jnp.int32),
            pltpu.VMEM((_BPW,), jnp.int32),
            pltpu.VMEM((_BPW,), jnp.int32),
            pltpu.VMEM((_BPW,), jnp.int32),
            pltpu.VMEM((_BPW,), jnp.float32),
        ] + set_scratch,
        compiler_params=pltpu.CompilerParams(needs_layout_passes=False,
                                             use_tc_tiling_on_sc=True),
    )
    return f(*idxs, ent3, rel2)


# speedup vs baseline: 2.1819x; 1.0087x over previous
import functools

import jax
import jax.numpy as jnp
from jax import lax
from jax.experimental import pallas as pl
from jax.experimental.pallas import tpu as pltpu
from jax.experimental.pallas import tpu_sc as plsc

_B = 16384
_D = 64
_NC = 2
_NS = 16
_NW = _NC * _NS
_BPW = _B // _NW
_G = 16
_NG = _BPW // _G
_NB = 3
_NQ = 10


def _body(ph, pr, pt, nh, nr, nt, ent3, rel2, pos_out, neg_out,
          rh, rr, rt, dr, out_v, *setargs):
    wid = lax.axis_index("s") * _NC + lax.axis_index("c")
    sets = [(setargs[3 * b], setargs[3 * b + 1], setargs[3 * b + 2])
            for b in range(_NB)]

    def fire(g, b):
        bht, brl, sem = sets[b]
        v_h = rh[pl.ds(g * _G, _G)]
        v_t = rt[pl.ds(g * _G, _G)]
        for i in range(_G):
            pltpu.async_copy(ent3.at[lax.shift_right_logical(v_h[i], 3)],
                             bht.at[i], sem)
            pltpu.async_copy(ent3.at[lax.shift_right_logical(v_t[i], 3)],
                             bht.at[i + _G], sem)
        pltpu.async_copy(rel2.at[dr.at[pl.ds(g * _G, _G)]], brl, sem)

    def drain(b):
        bht, brl, sem = sets[b]
        pltpu.make_async_copy(ent3.at[pl.ds(0, 2 * _G)], bht, sem).wait()
        pltpu.make_async_copy(rel2.at[pl.ds(0, _G)], brl, sem).wait()

    def compute(g, b):
        bht, brl, _ = sets[b]
        v_h = rh[pl.ds(g * _G, _G)]
        v_t = rt[pl.ds(g * _G, _G)]
        rows = lax.iota(jnp.int32, 16)
        k_h = v_h & 7
        k_t = v_t & 7
        base_r = (rr[pl.ds(g * _G, _G)] & 1) * _D

        rows_t = rows + _G

        def jbody(j8, acc):
            for u in range(8):
                j = j8 * 8 + u
                jv = lax.broadcast(j, (16,))
                h = plsc.load_gather(bht, [rows, k_h, jv])
                t = plsc.load_gather(bht, [rows_t, k_t, jv])
                r = plsc.load_gather(brl, [rows, base_r + j])
                d = h + r - t
                acc = acc + d * d
            return acc

        acc = lax.fori_loop(0, _D // 8, jbody, jnp.zeros((16,), jnp.float32))
        out_v[pl.ds(g * _G, 16)] = acc

    def do_term(hi, ri, ti, out_hbm):
        pltpu.sync_copy(hi.at[wid], rh)
        pltpu.sync_copy(ri.at[wid], rr)
        pltpu.sync_copy(ti.at[wid], rt)
        for s in range(_BPW // 16):
            v = rr[pl.ds(s * 16, 16)]
            dr[pl.ds(s * 16, 16)] = lax.shift_right_logical(v, 1)
        for b in range(_NB):
            fire(b, b)

        def qbody(q, _):
            for b in range(_NB):
                g = q * _NB + b
                drain(b)
                compute(g, b)

                @pl.when(g + _NB < _NG)
                def _():
                    fire(g + _NB, b)
            return 0

        lax.fori_loop(0, _NQ, qbody, 0)
        for g in range(_NQ * _NB, _NG):
            b = g % _NB
            drain(b)
            compute(g, b)
        pltpu.sync_copy(out_v, out_hbm.at[pl.ds(wid * _BPW, _BPW)])

    do_term(ph, pr, pt, pos_out)
    do_term(nh, nr, nt, neg_out)


@functools.partial(jax.jit)
def kernel(ph, pr, pt, nh, nr, nt, ent_embed, rel_embed):
    idxs = [x.astype(jnp.int32).reshape(_NW, _BPW)
            for x in (ph, pr, pt, nh, nr, nt)]
    ent3 = ent_embed.reshape(ent_embed.shape[0] // 8, 8, _D)
    rel2 = rel_embed.reshape(rel_embed.shape[0] // 2, 2 * _D)
    mesh = plsc.VectorSubcoreMesh(core_axis_name="c", subcore_axis_name="s",
                                  num_cores=_NC, num_subcores=_NS)
    set_scratch = []
    for _ in range(_NB):
        set_scratch += [
            pltpu.VMEM((2 * _G, 8, _D), jnp.float32),
            pltpu.VMEM((_G, 2 * _D), jnp.float32),
            pltpu.SemaphoreType.DMA,
        ]
    f = pl.kernel(
        _body,
        out_type=(jax.ShapeDtypeStruct((_B,), jnp.float32),
                  jax.ShapeDtypeStruct((_B,), jnp.float32)),
        mesh=mesh,
        scratch_types=[
            pltpu.VMEM((_BPW,), jnp.int32),
            pltpu.VMEM((_BPW,), jnp.int32),
            pltpu.VMEM((_BPW,), jnp.int32),
            pltpu.VMEM((_BPW,), jnp.int32),
            pltpu.VMEM((_BPW,), jnp.float32),
        ] + set_scratch,
        compiler_params=pltpu.CompilerParams(needs_layout_passes=False,
                                             use_tc_tiling_on_sc=True),
    )
    return f(*idxs, ent3, rel2)
